# Initial kernel scaffold; baseline (speedup 1.0000x reference)
import jax
import jax.numpy as jnp
from jax.experimental import pallas as pl


def kernel(x_atom, edge_index, edge_weight, edge_attr, mlp_W1, mlp_b1, mlp_W2, mlp_b2, lin1_W, lin2_W, lin2_b, lin_W, lin_b):
    raise NotImplementedError("write your pallas kernel here")


# speedup vs baseline: 1.4369x; 1.4369x over previous
import functools

import jax
import jax.numpy as jnp
from jax import lax
from jax.experimental import pallas as pl
from jax.experimental.pallas import tpu as pltpu
from jax.experimental.pallas import tpu_sc as plsc

N = 10000
E = 320000
H = 128
G = 50
F = 128
CUTOFF = 10.0
LOG2 = 0.6931471805599453

NC = 2
NS = 16
L = 16
NW = NC * NS
EPW = E // NW
CH = 80
NCHUNK = EPW // CH
RPT = 624
ZR = 208
NREM = N - NS * RPT


def _ssp(x):
    return jnp.maximum(x, 0.0) + jnp.log1p(jnp.exp(-jnp.abs(x))) - LOG2


BE = 2000


def _wf_body(ea_ref, ew_ref, w1_ref, b1_ref, w2_ref, b2_ref, out_ref):
    c = 0.5 * (jnp.cos(ew_ref[...] * (jnp.pi / CUTOFF)) + 1.0)
    h = _ssp(
        jnp.dot(ea_ref[...], w1_ref[...], preferred_element_type=jnp.float32)
        + b1_ref[...]
    )
    wf = (
        jnp.dot(h, w2_ref[...], preferred_element_type=jnp.float32)
        + b2_ref[...]
    )
    out_ref[...] = wf * c


def _wf_call(edge_attr, ew2, w1, b1r, w2, b2r):
    return pl.pallas_call(
        _wf_body,
        grid=(E // BE,),
        in_specs=[
            pl.BlockSpec((BE, G), lambda i: (i, 0)),
            pl.BlockSpec((BE, 1), lambda i: (i, 0)),
            pl.BlockSpec((G, F), lambda i: (0, 0)),
            pl.BlockSpec((1, F), lambda i: (0, 0)),
            pl.BlockSpec((F, F), lambda i: (0, 0)),
            pl.BlockSpec((1, F), lambda i: (0, 0)),
        ],
        out_specs=pl.BlockSpec((BE, F), lambda i: (i, 0)),
        out_shape=jax.ShapeDtypeStruct((E, F), jnp.float32),
    )(edge_attr, ew2, w1, b1r, w2, b2r)


def _xs_body(x_ref, w_ref, o_ref):
    o_ref[...] = jnp.dot(
        x_ref[...], w_ref[...], preferred_element_type=jnp.float32
    )


def _xs_call(x_atom, lin1_W):
    return pl.pallas_call(
        _xs_body,
        out_shape=jax.ShapeDtypeStruct((N, F), jnp.float32),
    )(x_atom, lin1_W)


@functools.cache
def _get_sc_scatter():
    mesh = plsc.VectorSubcoreMesh(
        core_axis_name="c", subcore_axis_name="s",
        num_cores=NC, num_subcores=NS,
    )
    return functools.partial(
        pl.kernel,
        mesh=mesh,
        out_type=jax.ShapeDtypeStruct((NC, N, F), jnp.float32),
        scratch_types=[
            pltpu.VMEM((CH,), jnp.int32),
            pltpu.VMEM((CH,), jnp.int32),
            pltpu.VMEM((CH, F), jnp.float32),
            pltpu.VMEM((CH, F), jnp.float32),
            pltpu.VMEM((ZR, F), jnp.float32),
            pltpu.VMEM_SHARED((N, F), jnp.float32),
            pltpu.SemaphoreType.DMA,
        ],
    )(_sc_scatter_body)


def _sc_scatter_body(
    xs_hbm, src_hbm, dst_hbm, wf_hbm, out_hbm,
    src_v, dst_v, gx_v, wf_v, st_v, agg_sh, sem,
):
    c = lax.axis_index("c")
    s = lax.axis_index("s")
    wid = s * NC + c

    def zrow(r, _):
        for j in range(F // L):
            st_v[r, pl.ds(j * L, L)] = jnp.zeros((L,), jnp.float32)
        return 0

    lax.fori_loop(0, ZR, zrow, 0)
    row0 = s * RPT
    for t in range(RPT // ZR):
        pltpu.sync_copy(st_v, agg_sh.at[pl.ds(row0 + t * ZR, ZR)])

    @pl.when(s == 0)
    def _zero_rem():
        pltpu.sync_copy(
            st_v.at[pl.ds(0, NREM)], agg_sh.at[pl.ds(NS * RPT, NREM)]
        )

    plsc.subcore_barrier()

    def chunk(k, _):
        base = wid * EPW + k * CH
        pltpu.sync_copy(src_hbm.at[pl.ds(base, CH)], src_v)
        pltpu.sync_copy(dst_hbm.at[pl.ds(base, CH)], dst_v)
        pltpu.async_copy(xs_hbm.at[src_v], gx_v, sem).wait()
        pltpu.sync_copy(wf_hbm.at[pl.ds(base, CH)], wf_v)

        def row(r, _):
            for j in range(F // L):
                sl = pl.ds(j * L, L)
                gx_v[r, sl] = gx_v[r, sl] * wf_v[r, sl]
            return 0

        lax.fori_loop(0, CH, row, 0)
        pltpu.sync_copy(gx_v, agg_sh.at[dst_v], add=True)
        return 0

    lax.fori_loop(0, NCHUNK, chunk, 0)
    plsc.subcore_barrier()

    for t in range(RPT // ZR):
        r0 = row0 + t * ZR
        pltpu.sync_copy(agg_sh.at[pl.ds(r0, ZR)], st_v)
        pltpu.sync_copy(st_v, out_hbm.at[c, pl.ds(r0, ZR)])

    @pl.when(s == 0)
    def _dump_rem():
        pltpu.sync_copy(
            agg_sh.at[pl.ds(NS * RPT, NREM)], st_v.at[pl.ds(0, NREM)]
        )
        pltpu.sync_copy(
            st_v.at[pl.ds(0, NREM)], out_hbm.at[c, pl.ds(NS * RPT, NREM)]
        )


BN = 2000


def _tail_body(p_ref, w2_ref, b2_ref, w_ref, b_ref, o_ref):
    agg = p_ref[0] + p_ref[1]
    t = _ssp(
        jnp.dot(agg, w2_ref[...], preferred_element_type=jnp.float32)
        + b2_ref[...]
    )
    o_ref[...] = (
        jnp.dot(t, w_ref[...], preferred_element_type=jnp.float32)
        + b_ref[...]
    )


def _tail_call(partials, lin2_W, lin2_br, lin_W, lin_br):
    return pl.pallas_call(
        _tail_body,
        grid=(N // BN,),
        in_specs=[
            pl.BlockSpec((NC, BN, F), lambda i: (0, i, 0)),
            pl.BlockSpec((F, H), lambda i: (0, 0)),
            pl.BlockSpec((1, H), lambda i: (0, 0)),
            pl.BlockSpec((H, H), lambda i: (0, 0)),
            pl.BlockSpec((1, H), lambda i: (0, 0)),
        ],
        out_specs=pl.BlockSpec((BN, H), lambda i: (i, 0)),
        out_shape=jax.ShapeDtypeStruct((N, H), jnp.float32),
    )(partials, lin2_W, lin2_br, lin_W, lin_br)


@jax.jit
def kernel(
    x_atom, edge_index, edge_weight, edge_attr,
    mlp_W1, mlp_b1, mlp_W2, mlp_b2,
    lin1_W, lin2_W, lin2_b, lin_W, lin_b,
):
    ew2 = edge_weight.reshape(E, 1)
    wf = _wf_call(
        edge_attr, ew2, mlp_W1, mlp_b1.reshape(1, F), mlp_W2,
        mlp_b2.reshape(1, F),
    )
    xs = _xs_call(x_atom, lin1_W)
    src = edge_index[0]
    dst = edge_index[1]
    partials = _get_sc_scatter()(xs, src, dst, wf)
    return _tail_call(
        partials, lin2_W, lin2_b.reshape(1, H), lin_W, lin_b.reshape(1, H)
    )


# speedup vs baseline: 2.6850x; 1.8686x over previous
import functools

import jax
import jax.numpy as jnp
from jax import lax
from jax.experimental import pallas as pl
from jax.experimental.pallas import tpu as pltpu
from jax.experimental.pallas import tpu_sc as plsc

N = 10000
E = 320000
H = 128
G = 50
F = 128
CUTOFF = 10.0
LOG2 = 0.6931471805599453

NC = 2
NS = 16
L = 16
NW = NC * NS
CH = 128
NCID = E // CH
NQ = NCID // NW
NEXTRA = NCID - NQ * NW
RPT = 624
ZR = 104
NREM = N - NS * RPT


def _ssp(x):
    return jnp.maximum(x, 0.0) + jnp.log1p(jnp.exp(-jnp.abs(x))) - LOG2


BE = 2560
NBC = BE // 128


def _wf_body(ea_ref, ew_ref, w1_ref, b1_ref, w2_ref, b2_ref, out_ref):
    cpk = 0.5 * (jnp.cos(ew_ref[0] * (jnp.pi / CUTOFF)) + 1.0)
    ct = jnp.swapaxes(cpk, 0, 1)
    h = _ssp(
        jnp.dot(ea_ref[...], w1_ref[...], preferred_element_type=jnp.float32)
        + b1_ref[...]
    )
    wf = (
        jnp.dot(h, w2_ref[...], preferred_element_type=jnp.float32)
        + b2_ref[...]
    )
    for a in range(NBC):
        out_ref[a * 128:(a + 1) * 128, :] = (
            wf[a * 128:(a + 1) * 128, :] * ct[:, a:a + 1]
        )


def _wf_call(edge_attr, ewp, w1, b1r, w2, b2r):
    return pl.pallas_call(
        _wf_body,
        grid=(E // BE,),
        in_specs=[
            pl.BlockSpec((BE, G), lambda i: (i, 0)),
            pl.BlockSpec((1, NBC, 128), lambda i: (i, 0, 0)),
            pl.BlockSpec((G, F), lambda i: (0, 0)),
            pl.BlockSpec((1, F), lambda i: (0, 0)),
            pl.BlockSpec((F, F), lambda i: (0, 0)),
            pl.BlockSpec((1, F), lambda i: (0, 0)),
        ],
        out_specs=pl.BlockSpec((BE, F), lambda i: (i, 0)),
        out_shape=jax.ShapeDtypeStruct((E, F), jnp.float32),
    )(edge_attr, ewp, w1, b1r, w2, b2r)


def _xs_body(x_ref, w_ref, o_ref):
    o_ref[...] = jnp.dot(
        x_ref[...], w_ref[...], preferred_element_type=jnp.float32
    )


def _xs_call(x_atom, lin1_W):
    return pl.pallas_call(
        _xs_body,
        out_shape=jax.ShapeDtypeStruct((N, F), jnp.float32),
    )(x_atom, lin1_W)


@functools.cache
def _get_sc_scatter():
    mesh = plsc.VectorSubcoreMesh(
        core_axis_name="c", subcore_axis_name="s",
        num_cores=NC, num_subcores=NS,
    )
    return functools.partial(
        pl.kernel,
        mesh=mesh,
        out_type=jax.ShapeDtypeStruct((NC, N, F), jnp.float32),
        scratch_types=[
            pltpu.VMEM((CH,), jnp.int32),
            pltpu.VMEM((CH,), jnp.int32),
            pltpu.VMEM((CH, F), jnp.float32),
            pltpu.VMEM((CH, F), jnp.float32),
            pltpu.VMEM((ZR, F), jnp.float32),
            pltpu.VMEM_SHARED((N, F), jnp.float32),
            pltpu.SemaphoreType.DMA,
        ],
    )(_sc_scatter_body)


def _sc_scatter_body(
    xs_hbm, src_hbm, dst_hbm, wf_hbm, out_hbm,
    src_v, dst_v, gx_v, wf_v, st_v, agg_sh, sem,
):
    c = lax.axis_index("c")
    s = lax.axis_index("s")
    wid = s * NC + c

    def zrow(r, _):
        for j in range(F // L):
            st_v[r, pl.ds(j * L, L)] = jnp.zeros((L,), jnp.float32)
        return 0

    lax.fori_loop(0, ZR, zrow, 0)
    row0 = s * RPT
    for t in range(RPT // ZR):
        pltpu.sync_copy(st_v, agg_sh.at[pl.ds(row0 + t * ZR, ZR)])

    @pl.when(s == 0)
    def _zero_rem():
        pltpu.sync_copy(
            st_v.at[pl.ds(0, NREM)], agg_sh.at[pl.ds(NS * RPT, NREM)]
        )

    plsc.subcore_barrier()

    def chunk(cid):
        base = cid * CH
        pltpu.sync_copy(src_hbm.at[pl.ds(base, CH)], src_v)
        pltpu.sync_copy(dst_hbm.at[pl.ds(base, CH)], dst_v)
        pltpu.async_copy(xs_hbm.at[src_v], gx_v, sem).wait()
        pltpu.sync_copy(wf_hbm.at[pl.ds(base, CH)], wf_v)

        def row(r, _):
            for j in range(F // L):
                sl = (r, pl.ds(j * L, L))
                gx_v[sl] = gx_v[sl] * wf_v[sl]
            return 0

        lax.fori_loop(0, CH, row, 0)
        pltpu.sync_copy(gx_v, agg_sh.at[dst_v], add=True)

    def round_body(q, _):
        chunk(q * NW + wid)
        return 0

    lax.fori_loop(0, NQ, round_body, 0)

    @pl.when(wid < NEXTRA)
    def _extra():
        chunk(NQ * NW + wid)

    plsc.subcore_barrier()

    for t in range(RPT // ZR):
        r0 = row0 + t * ZR
        pltpu.sync_copy(agg_sh.at[pl.ds(r0, ZR)], st_v)
        pltpu.sync_copy(st_v, out_hbm.at[c, pl.ds(r0, ZR)])

    @pl.when(s == 0)
    def _dump_rem():
        pltpu.sync_copy(
            agg_sh.at[pl.ds(NS * RPT, NREM)], st_v.at[pl.ds(0, NREM)]
        )
        pltpu.sync_copy(
            st_v.at[pl.ds(0, NREM)], out_hbm.at[c, pl.ds(NS * RPT, NREM)]
        )


BN = 2000


def _tail_body(p_ref, w2_ref, b2_ref, w_ref, b_ref, o_ref):
    agg = p_ref[0] + p_ref[1]
    t = _ssp(
        jnp.dot(agg, w2_ref[...], preferred_element_type=jnp.float32)
        + b2_ref[...]
    )
    o_ref[...] = (
        jnp.dot(t, w_ref[...], preferred_element_type=jnp.float32)
        + b_ref[...]
    )


def _tail_call(partials, lin2_W, lin2_br, lin_W, lin_br):
    return pl.pallas_call(
        _tail_body,
        grid=(N // BN,),
        in_specs=[
            pl.BlockSpec((NC, BN, F), lambda i: (0, i, 0)),
            pl.BlockSpec((F, H), lambda i: (0, 0)),
            pl.BlockSpec((1, H), lambda i: (0, 0)),
            pl.BlockSpec((H, H), lambda i: (0, 0)),
            pl.BlockSpec((1, H), lambda i: (0, 0)),
        ],
        out_specs=pl.BlockSpec((BN, H), lambda i: (i, 0)),
        out_shape=jax.ShapeDtypeStruct((N, H), jnp.float32),
    )(partials, lin2_W, lin2_br, lin_W, lin_br)


@jax.jit
def kernel(
    x_atom, edge_index, edge_weight, edge_attr,
    mlp_W1, mlp_b1, mlp_W2, mlp_b2,
    lin1_W, lin2_W, lin2_b, lin_W, lin_b,
):
    wf = _wf_call(
        edge_attr, edge_weight.reshape(E // BE, NBC, 128),
        mlp_W1, mlp_b1.reshape(1, F), mlp_W2, mlp_b2.reshape(1, F),
    )
    xs = _xs_call(x_atom, lin1_W)
    src = edge_index[0]
    dst = edge_index[1]
    partials = _get_sc_scatter()(xs, src, dst, wf)
    return _tail_call(
        partials, lin2_W, lin2_b.reshape(1, H), lin_W, lin_b.reshape(1, H)
    )


# speedup vs baseline: 3.7794x; 1.4076x over previous
import functools

import jax
import jax.numpy as jnp
from jax import lax
from jax.experimental import pallas as pl
from jax.experimental.pallas import tpu as pltpu
from jax.experimental.pallas import tpu_sc as plsc

N = 10000
E = 320000
H = 128
G = 50
F = 128
CUTOFF = 10.0
LOG2 = 0.6931471805599453

NC = 2
NS = 16
L = 16
NW = NC * NS
CH = 80
EPW = E // NW
NCHUNKS = EPW // CH
NEI = E // CH
RPT = 624
NREM = N - NS * RPT


def _ssp(x):
    return jnp.maximum(x, 0.0) + jnp.log1p(jnp.exp(-jnp.abs(x))) - LOG2


BE = 2560
NBC = BE // 128


def _wf_body(ea_ref, ew_ref, w1_ref, b1_ref, w2_ref, b2_ref, out_ref):
    cpk = 0.5 * (jnp.cos(ew_ref[0] * (jnp.pi / CUTOFF)) + 1.0)
    ct = jnp.swapaxes(cpk, 0, 1)
    h = _ssp(
        jnp.dot(ea_ref[...], w1_ref[...], preferred_element_type=jnp.float32)
        + b1_ref[...]
    )
    wf = (
        jnp.dot(h, w2_ref[...], preferred_element_type=jnp.float32)
        + b2_ref[...]
    )
    for a in range(NBC):
        out_ref[a * 128:(a + 1) * 128, :] = (
            wf[a * 128:(a + 1) * 128, :] * ct[:, a:a + 1]
        )


def _wf_call(edge_attr, ewp, w1, b1r, w2, b2r):
    return pl.pallas_call(
        _wf_body,
        grid=(E // BE,),
        in_specs=[
            pl.BlockSpec((BE, G), lambda i: (i, 0)),
            pl.BlockSpec((1, NBC, 128), lambda i: (i, 0, 0)),
            pl.BlockSpec((G, F), lambda i: (0, 0)),
            pl.BlockSpec((1, F), lambda i: (0, 0)),
            pl.BlockSpec((F, F), lambda i: (0, 0)),
            pl.BlockSpec((1, F), lambda i: (0, 0)),
        ],
        out_specs=pl.BlockSpec((BE, F), lambda i: (i, 0)),
        out_shape=jax.ShapeDtypeStruct((E, F), jnp.float32),
    )(edge_attr, ewp, w1, b1r, w2, b2r)


def _xs_body(x_ref, w_ref, o_ref):
    o_ref[...] = jnp.dot(
        x_ref[...], w_ref[...], preferred_element_type=jnp.float32
    )


def _xs_call(x_atom, lin1_W):
    return pl.pallas_call(
        _xs_body,
        out_shape=jax.ShapeDtypeStruct((N, F), jnp.float32),
    )(x_atom, lin1_W)


@functools.cache
def _get_sc_scatter():
    mesh = plsc.VectorSubcoreMesh(
        core_axis_name="c", subcore_axis_name="s",
        num_cores=NC, num_subcores=NS,
    )
    return functools.partial(
        pl.kernel,
        mesh=mesh,
        out_type=jax.ShapeDtypeStruct((NC, N, F), jnp.float32),
        scratch_types=[
            pltpu.VMEM((2, CH), jnp.int32),
            pltpu.VMEM((2, CH), jnp.int32),
            pltpu.VMEM((2, CH), jnp.int32),
            pltpu.VMEM((2, CH), jnp.int32),
            pltpu.VMEM((CH, F), jnp.float32),
            pltpu.VMEM((CH, F), jnp.float32),
            pltpu.VMEM((CH, F), jnp.float32),
            pltpu.VMEM((CH, F), jnp.float32),
            pltpu.VMEM_SHARED((N, F), jnp.float32),
            pltpu.SemaphoreType.DMA,
            pltpu.SemaphoreType.DMA,
            pltpu.SemaphoreType.DMA,
            pltpu.SemaphoreType.DMA,
            pltpu.SemaphoreType.DMA,
            pltpu.SemaphoreType.DMA,
            pltpu.SemaphoreType.DMA,
            pltpu.SemaphoreType.DMA,
            pltpu.SemaphoreType.DMA,
            pltpu.SemaphoreType.DMA,
        ],
    )(_sc_scatter_body)


def _sc_scatter_body(
    xs_hbm, ei_hbm, wf_hbm, out_hbm,
    ix0, ix1, ix2, ix3, gx0, gx1, wfv0, wfv1, agg_sh,
    is0, is1, is2, is3, gs0, gs1, ws0, ws1, ss0, ss1,
):
    c = lax.axis_index("c")
    s = lax.axis_index("s")
    wid = s * NC + c
    cid0 = wid * NCHUNKS

    ixv = [ix0, ix1, ix2, ix3]
    gxv = [gx0, gx1]
    wfv = [wfv0, wfv1]
    isem = [is0, is1, is2, is3]
    gsem = [gs0, gs1]
    wsem = [ws0, ws1]
    ssem = [ss0, ss1]

    def issue_idx(t, p):
        pltpu.async_copy(ei_hbm.at[cid0 + t], ixv[p], isem[p])

    def wait_idx(p):
        pltpu.make_async_copy(ei_hbm.at[0], ixv[p], isem[p]).wait()

    def issue_gw(t, p, b):
        base = (cid0 + t) * CH
        pltpu.async_copy(xs_hbm.at[ixv[p].at[0]], gxv[b], gsem[b])
        pltpu.async_copy(wf_hbm.at[pl.ds(base, CH)], wfv[b], wsem[b])

    def wait_gw(b):
        pltpu.make_async_copy(xs_hbm.at[pl.ds(0, CH)], gxv[b], gsem[b]).wait()
        pltpu.make_async_copy(wf_hbm.at[pl.ds(0, CH)], wfv[b], wsem[b]).wait()

    def issue_scatter(p, b):
        pltpu.async_copy(gxv[b], agg_sh.at[ixv[p].at[1]], ssem[b], add=True)

    def wait_scatter(b):
        pltpu.make_async_copy(
            gxv[b], agg_sh.at[pl.ds(0, CH)], ssem[b]
        ).wait()

    def multiply(b):
        def row(r, _):
            for j in range(F // L):
                sl = (r, pl.ds(j * L, L))
                gxv[b][sl] = gxv[b][sl] * wfv[b][sl]
            return 0

        lax.fori_loop(0, CH, row, 0)

    def zrow(r, _):
        for j in range(F // L):
            gx0[r, pl.ds(j * L, L)] = jnp.zeros((L,), jnp.float32)
        return 0

    lax.fori_loop(0, CH, zrow, 0)
    row0 = s * RPT
    for t in range(7):
        pltpu.sync_copy(gx0, agg_sh.at[pl.ds(row0 + t * CH, CH)])
    pltpu.sync_copy(
        gx0.at[pl.ds(0, 64)], agg_sh.at[pl.ds(row0 + 7 * CH, 64)]
    )

    @pl.when(s == 0)
    def _zero_rem():
        pltpu.sync_copy(
            gx0.at[pl.ds(0, NREM)], agg_sh.at[pl.ds(NS * RPT, NREM)]
        )

    plsc.subcore_barrier()

    def substep(t, p):
        b = p & 1
        nb = (p + 1) & 1
        np_ = (p + 1) & 3
        wait_gw(b)

        @pl.when(t + 2 < NCHUNKS)
        def _():
            issue_idx(t + 2, (p + 2) & 3)

        @pl.when(t + 1 < NCHUNKS)
        def _():
            @pl.when(t >= 1)
            def _():
                wait_scatter(nb)

            wait_idx(np_)
            issue_gw(t + 1, np_, nb)

        multiply(b)
        issue_scatter(p, b)

    issue_idx(0, 0)
    issue_idx(1, 1)
    wait_idx(0)
    issue_gw(0, 0, 0)

    def quad(q, _):
        for p in range(4):
            substep(q * 4 + p, p)
        return 0

    lax.fori_loop(0, NCHUNKS // 4, quad, 0)
    substep(NCHUNKS - 1, 0)
    wait_scatter(0)
    wait_scatter(1)

    plsc.subcore_barrier()

    for t in range(7):
        r0 = row0 + t * CH
        pltpu.sync_copy(agg_sh.at[pl.ds(r0, CH)], gx0)
        pltpu.sync_copy(gx0, out_hbm.at[c, pl.ds(r0, CH)])
    pltpu.sync_copy(
        agg_sh.at[pl.ds(row0 + 7 * CH, 64)], gx0.at[pl.ds(0, 64)]
    )
    pltpu.sync_copy(gx0.at[pl.ds(0, 64)], out_hbm.at[c, pl.ds(row0 + 7 * CH, 64)])

    @pl.when(s == 0)
    def _dump_rem():
        pltpu.sync_copy(
            agg_sh.at[pl.ds(NS * RPT, NREM)], gx1.at[pl.ds(0, NREM)]
        )
        pltpu.sync_copy(
            gx1.at[pl.ds(0, NREM)], out_hbm.at[c, pl.ds(NS * RPT, NREM)]
        )


BN = 2000


def _tail_body(p_ref, w2_ref, b2_ref, w_ref, b_ref, o_ref):
    agg = p_ref[0] + p_ref[1]
    t = _ssp(
        jnp.dot(agg, w2_ref[...], preferred_element_type=jnp.float32)
        + b2_ref[...]
    )
    o_ref[...] = (
        jnp.dot(t, w_ref[...], preferred_element_type=jnp.float32)
        + b_ref[...]
    )


def _tail_call(partials, lin2_W, lin2_br, lin_W, lin_br):
    return pl.pallas_call(
        _tail_body,
        grid=(N // BN,),
        in_specs=[
            pl.BlockSpec((NC, BN, F), lambda i: (0, i, 0)),
            pl.BlockSpec((F, H), lambda i: (0, 0)),
            pl.BlockSpec((1, H), lambda i: (0, 0)),
            pl.BlockSpec((H, H), lambda i: (0, 0)),
            pl.BlockSpec((1, H), lambda i: (0, 0)),
        ],
        out_specs=pl.BlockSpec((BN, H), lambda i: (i, 0)),
        out_shape=jax.ShapeDtypeStruct((N, H), jnp.float32),
    )(partials, lin2_W, lin2_br, lin_W, lin_br)


@jax.jit
def kernel(
    x_atom, edge_index, edge_weight, edge_attr,
    mlp_W1, mlp_b1, mlp_W2, mlp_b2,
    lin1_W, lin2_W, lin2_b, lin_W, lin_b,
):
    wf = _wf_call(
        edge_attr, edge_weight.reshape(E // BE, NBC, 128),
        mlp_W1, mlp_b1.reshape(1, F), mlp_W2, mlp_b2.reshape(1, F),
    )
    xs = _xs_call(x_atom, lin1_W)
    ei_r = jnp.stack(
        [edge_index[0].reshape(NEI, CH), edge_index[1].reshape(NEI, CH)],
        axis=1,
    )
    partials = _get_sc_scatter()(xs, ei_r, wf)
    return _tail_call(
        partials, lin2_W, lin2_b.reshape(1, H), lin_W, lin_b.reshape(1, H)
    )


# speedup vs baseline: 4.8338x; 1.2790x over previous
import functools

import jax
import jax.numpy as jnp
from jax import lax
from jax.experimental import pallas as pl
from jax.experimental.pallas import tpu as pltpu
from jax.experimental.pallas import tpu_sc as plsc

N = 10000
E = 320000
H = 128
G = 50
F = 128
CUTOFF = 10.0
LOG2 = 0.6931471805599453

NC = 2
NS = 16
L = 16
NW = NC * NS
CH = 80
EPW = E // NW
NCHUNKS = EPW // CH
NEI = E // CH
RPT = 624
NREM = N - NS * RPT


def _ssp(x):
    return jnp.maximum(x, 0.0) + jnp.log1p(jnp.exp(-jnp.abs(x))) - LOG2


BE = 2560
NBC = BE // 128


def _wf_body(ea_ref, ew_ref, w1_ref, b1_ref, w2_ref, b2_ref, out_ref):
    cpk = 0.5 * (jnp.cos(ew_ref[0] * (jnp.pi / CUTOFF)) + 1.0)
    ct = jnp.swapaxes(cpk, 0, 1)
    h = _ssp(
        lax.dot_general(
            ea_ref[...], w1_ref[...],
            (((0,), (0,)), ((), ())),
            preferred_element_type=jnp.float32,
        )
        + b1_ref[...]
    )
    wf = (
        jnp.dot(h, w2_ref[...], preferred_element_type=jnp.float32)
        + b2_ref[...]
    )
    for a in range(NBC):
        out_ref[a * 128:(a + 1) * 128, :] = (
            wf[a * 128:(a + 1) * 128, :] * ct[:, a:a + 1]
        )


def _wf_call(ea_t, ewp, w1, b1r, w2, b2r):
    return pl.pallas_call(
        _wf_body,
        grid=(E // BE,),
        in_specs=[
            pl.BlockSpec((G, BE), lambda i: (0, i)),
            pl.BlockSpec((1, NBC, 128), lambda i: (i, 0, 0)),
            pl.BlockSpec((G, F), lambda i: (0, 0)),
            pl.BlockSpec((1, F), lambda i: (0, 0)),
            pl.BlockSpec((F, F), lambda i: (0, 0)),
            pl.BlockSpec((1, F), lambda i: (0, 0)),
        ],
        out_specs=pl.BlockSpec((BE, F), lambda i: (i, 0)),
        out_shape=jax.ShapeDtypeStruct((E, F), jnp.float32),
    )(ea_t, ewp, w1, b1r, w2, b2r)


def _xs_body(x_ref, w_ref, o_ref):
    o_ref[...] = jnp.dot(
        x_ref[...], w_ref[...], preferred_element_type=jnp.float32
    )


def _xs_call(x_atom, lin1_W):
    return pl.pallas_call(
        _xs_body,
        out_shape=jax.ShapeDtypeStruct((N, F), jnp.float32),
    )(x_atom, lin1_W)


@functools.cache
def _get_sc_scatter():
    mesh = plsc.VectorSubcoreMesh(
        core_axis_name="c", subcore_axis_name="s",
        num_cores=NC, num_subcores=NS,
    )
    return functools.partial(
        pl.kernel,
        mesh=mesh,
        out_type=jax.ShapeDtypeStruct((NC, N, F), jnp.float32),
        scratch_types=[
            pltpu.VMEM((2, CH), jnp.int32),
            pltpu.VMEM((2, CH), jnp.int32),
            pltpu.VMEM((2, CH), jnp.int32),
            pltpu.VMEM((2, CH), jnp.int32),
            pltpu.VMEM((CH, F), jnp.float32),
            pltpu.VMEM((CH, F), jnp.float32),
            pltpu.VMEM((CH, F), jnp.float32),
            pltpu.VMEM((CH, F), jnp.float32),
            pltpu.VMEM_SHARED((N, F), jnp.float32),
            pltpu.SemaphoreType.DMA,
            pltpu.SemaphoreType.DMA,
            pltpu.SemaphoreType.DMA,
            pltpu.SemaphoreType.DMA,
            pltpu.SemaphoreType.DMA,
            pltpu.SemaphoreType.DMA,
            pltpu.SemaphoreType.DMA,
            pltpu.SemaphoreType.DMA,
            pltpu.SemaphoreType.DMA,
            pltpu.SemaphoreType.DMA,
        ],
    )(_sc_scatter_body)


def _sc_scatter_body(
    xs_hbm, ei_hbm, wf_hbm, out_hbm,
    ix0, ix1, ix2, ix3, gx0, gx1, wfv0, wfv1, agg_sh,
    is0, is1, is2, is3, gs0, gs1, ws0, ws1, ss0, ss1,
):
    c = lax.axis_index("c")
    s = lax.axis_index("s")
    wid = s * NC + c
    cid0 = wid * NCHUNKS

    ixv = [ix0, ix1, ix2, ix3]
    gxv = [gx0, gx1]
    wfv = [wfv0, wfv1]
    isem = [is0, is1, is2, is3]
    gsem = [gs0, gs1]
    wsem = [ws0, ws1]
    ssem = [ss0, ss1]

    def issue_idx(t, p):
        pltpu.async_copy(ei_hbm.at[cid0 + t], ixv[p], isem[p])

    def wait_idx(p):
        pltpu.make_async_copy(ei_hbm.at[0], ixv[p], isem[p]).wait()

    def issue_gw(t, p, b):
        base = (cid0 + t) * CH
        pltpu.async_copy(xs_hbm.at[ixv[p].at[0]], gxv[b], gsem[b])
        pltpu.async_copy(wf_hbm.at[pl.ds(base, CH)], wfv[b], wsem[b])

    def wait_gw(b):
        pltpu.make_async_copy(xs_hbm.at[pl.ds(0, CH)], gxv[b], gsem[b]).wait()
        pltpu.make_async_copy(wf_hbm.at[pl.ds(0, CH)], wfv[b], wsem[b]).wait()

    def issue_scatter(p, b):
        pltpu.async_copy(gxv[b], agg_sh.at[ixv[p].at[1]], ssem[b], add=True)

    def wait_scatter(b):
        pltpu.make_async_copy(
            gxv[b], agg_sh.at[pl.ds(0, CH)], ssem[b]
        ).wait()

    def multiply(b):
        def row(r, _):
            for j in range(F // L):
                sl = (r, pl.ds(j * L, L))
                gxv[b][sl] = gxv[b][sl] * wfv[b][sl]
            return 0

        lax.fori_loop(0, CH, row, 0)

    def zrow(r, _):
        for j in range(F // L):
            gx0[r, pl.ds(j * L, L)] = jnp.zeros((L,), jnp.float32)
        return 0

    lax.fori_loop(0, CH, zrow, 0)
    row0 = s * RPT
    for t in range(7):
        pltpu.sync_copy(gx0, agg_sh.at[pl.ds(row0 + t * CH, CH)])
    pltpu.sync_copy(
        gx0.at[pl.ds(0, 64)], agg_sh.at[pl.ds(row0 + 7 * CH, 64)]
    )

    @pl.when(s == 0)
    def _zero_rem():
        pltpu.sync_copy(
            gx0.at[pl.ds(0, NREM)], agg_sh.at[pl.ds(NS * RPT, NREM)]
        )

    plsc.subcore_barrier()

    def substep(t, p):
        b = p & 1
        nb = (p + 1) & 1
        np_ = (p + 1) & 3
        wait_gw(b)

        @pl.when(t + 2 < NCHUNKS)
        def _():
            issue_idx(t + 2, (p + 2) & 3)

        @pl.when(t + 1 < NCHUNKS)
        def _():
            @pl.when(t >= 1)
            def _():
                wait_scatter(nb)

            wait_idx(np_)
            issue_gw(t + 1, np_, nb)

        multiply(b)
        issue_scatter(p, b)

    issue_idx(0, 0)
    issue_idx(1, 1)
    wait_idx(0)
    issue_gw(0, 0, 0)

    def quad(q, _):
        for p in range(4):
            substep(q * 4 + p, p)
        return 0

    lax.fori_loop(0, NCHUNKS // 4, quad, 0)
    substep(NCHUNKS - 1, 0)
    wait_scatter(0)
    wait_scatter(1)

    plsc.subcore_barrier()

    for t in range(7):
        r0 = row0 + t * CH
        pltpu.sync_copy(agg_sh.at[pl.ds(r0, CH)], gx0)
        pltpu.sync_copy(gx0, out_hbm.at[c, pl.ds(r0, CH)])
    pltpu.sync_copy(
        agg_sh.at[pl.ds(row0 + 7 * CH, 64)], gx0.at[pl.ds(0, 64)]
    )
    pltpu.sync_copy(gx0.at[pl.ds(0, 64)], out_hbm.at[c, pl.ds(row0 + 7 * CH, 64)])

    @pl.when(s == 0)
    def _dump_rem():
        pltpu.sync_copy(
            agg_sh.at[pl.ds(NS * RPT, NREM)], gx1.at[pl.ds(0, NREM)]
        )
        pltpu.sync_copy(
            gx1.at[pl.ds(0, NREM)], out_hbm.at[c, pl.ds(NS * RPT, NREM)]
        )


BN = 2000


def _tail_body(p_ref, w2_ref, b2_ref, w_ref, b_ref, o_ref):
    agg = p_ref[0] + p_ref[1]
    t = _ssp(
        jnp.dot(agg, w2_ref[...], preferred_element_type=jnp.float32)
        + b2_ref[...]
    )
    o_ref[...] = (
        jnp.dot(t, w_ref[...], preferred_element_type=jnp.float32)
        + b_ref[...]
    )


def _tail_call(partials, lin2_W, lin2_br, lin_W, lin_br):
    return pl.pallas_call(
        _tail_body,
        grid=(N // BN,),
        in_specs=[
            pl.BlockSpec((NC, BN, F), lambda i: (0, i, 0)),
            pl.BlockSpec((F, H), lambda i: (0, 0)),
            pl.BlockSpec((1, H), lambda i: (0, 0)),
            pl.BlockSpec((H, H), lambda i: (0, 0)),
            pl.BlockSpec((1, H), lambda i: (0, 0)),
        ],
        out_specs=pl.BlockSpec((BN, H), lambda i: (i, 0)),
        out_shape=jax.ShapeDtypeStruct((N, H), jnp.float32),
    )(partials, lin2_W, lin2_br, lin_W, lin_br)


@jax.jit
def kernel(
    x_atom, edge_index, edge_weight, edge_attr,
    mlp_W1, mlp_b1, mlp_W2, mlp_b2,
    lin1_W, lin2_W, lin2_b, lin_W, lin_b,
):
    wf = _wf_call(
        edge_attr.T, edge_weight.reshape(E // BE, NBC, 128),
        mlp_W1, mlp_b1.reshape(1, F), mlp_W2, mlp_b2.reshape(1, F),
    )
    xs = _xs_call(x_atom, lin1_W)
    ei_r = jnp.stack(
        [edge_index[0].reshape(NEI, CH), edge_index[1].reshape(NEI, CH)],
        axis=1,
    )
    partials = _get_sc_scatter()(xs, ei_r, wf)
    return _tail_call(
        partials, lin2_W, lin2_b.reshape(1, H), lin_W, lin_b.reshape(1, H)
    )


# speedup vs baseline: 4.9688x; 1.0279x over previous
import functools

import jax
import jax.numpy as jnp
import numpy as np
from jax import lax
from jax.experimental import pallas as pl
from jax.experimental.pallas import tpu as pltpu
from jax.experimental.pallas import tpu_sc as plsc

N = 10000
E = 320000
H = 128
G = 50
F = 128
CUTOFF = 10.0
LOG2 = 0.6931471805599453

NC = 2
NS = 16
L = 16
NW = NC * NS
CH = 40
CPB = 8
NBLK0 = 31
NEI = E // (CH * CPB)
RPT = 624
NREM = N - NS * RPT

_TAU = np.empty((F,), dtype=np.int32)
for _jj in range(4):
    for _i in range(16):
        _TAU[32 * _jj + 2 * _i] = 32 * _jj + _i
        _TAU[32 * _jj + 2 * _i + 1] = 32 * _jj + 16 + _i


def _ssp(x):
    return jnp.maximum(x, 0.0) + jnp.log1p(jnp.exp(-jnp.abs(x))) - LOG2


BE = 2560
NBC = BE // 128


def _wf_body(ea_ref, ew_ref, w1_ref, b1_ref, w2_ref, b2_ref, out_ref):
    cpk = 0.5 * (jnp.cos(ew_ref[0] * (jnp.pi / CUTOFF)) + 1.0)
    ct = jnp.swapaxes(cpk, 0, 1)
    h = _ssp(
        lax.dot_general(
            ea_ref[...], w1_ref[...],
            (((0,), (0,)), ((), ())),
            preferred_element_type=jnp.float32,
        )
        + b1_ref[...]
    )
    wf = (
        jnp.dot(h, w2_ref[...], preferred_element_type=jnp.float32)
        + b2_ref[...]
    )
    for a in range(NBC):
        out_ref[a * 128:(a + 1) * 128, :] = (
            wf[a * 128:(a + 1) * 128, :] * ct[:, a:a + 1]
        )


def _wf_call(ea_t, ewp, w1, b1r, w2, b2r):
    return pl.pallas_call(
        _wf_body,
        grid=(E // BE,),
        in_specs=[
            pl.BlockSpec((G, BE), lambda i: (0, i)),
            pl.BlockSpec((1, NBC, 128), lambda i: (i, 0, 0)),
            pl.BlockSpec((G, F), lambda i: (0, 0)),
            pl.BlockSpec((1, F), lambda i: (0, 0)),
            pl.BlockSpec((F, F), lambda i: (0, 0)),
            pl.BlockSpec((1, F), lambda i: (0, 0)),
        ],
        out_specs=pl.BlockSpec((BE, F), lambda i: (i, 0)),
        out_shape=jax.ShapeDtypeStruct((E, F), jnp.float32),
    )(ea_t, ewp, w1, b1r, w2, b2r)


def _xs_body(x_ref, w_ref, o_ref):
    o_ref[...] = jnp.dot(
        x_ref[...], w_ref[...], preferred_element_type=jnp.float32
    )


def _xs_call(x_atom, lin1_W):
    return pl.pallas_call(
        _xs_body,
        out_shape=jax.ShapeDtypeStruct((N, F), jnp.float32),
    )(x_atom, lin1_W)


@functools.cache
def _get_sc_scatter():
    mesh = plsc.VectorSubcoreMesh(
        core_axis_name="c", subcore_axis_name="s",
        num_cores=NC, num_subcores=NS,
    )
    return functools.partial(
        pl.kernel,
        mesh=mesh,
        out_type=jax.ShapeDtypeStruct((NC, N, F), jnp.float32),
        scratch_types=[
            pltpu.VMEM((2 * CPB, CH), jnp.int32),
            pltpu.VMEM((2 * CPB, CH), jnp.int32),
            pltpu.VMEM((CH, F), jnp.float32),
            pltpu.VMEM((CH, F), jnp.float32),
            pltpu.VMEM((CH, F), jnp.float32),
            pltpu.VMEM((CH, F), jnp.float32),
            pltpu.VMEM((CH, F), jnp.float32),
            pltpu.VMEM((CH, F), jnp.float32),
            pltpu.VMEM((CH, F), jnp.float32),
            pltpu.VMEM((CH, F), jnp.float32),
            pltpu.VMEM_SHARED((N, F), jnp.float32),
            pltpu.SemaphoreType.DMA,
            pltpu.SemaphoreType.DMA,
            pltpu.SemaphoreType.DMA,
            pltpu.SemaphoreType.DMA,
            pltpu.SemaphoreType.DMA,
            pltpu.SemaphoreType.DMA,
            pltpu.SemaphoreType.DMA,
            pltpu.SemaphoreType.DMA,
            pltpu.SemaphoreType.DMA,
            pltpu.SemaphoreType.DMA,
            pltpu.SemaphoreType.DMA,
            pltpu.SemaphoreType.DMA,
            pltpu.SemaphoreType.DMA,
            pltpu.SemaphoreType.DMA,
        ],
    )(_sc_scatter_body)


def _sc_scatter_body(
    xs_hbm, ei_hbm, wf_hbm, out_hbm,
    ib0, ib1, gx0, gx1, gx2, gx3, wv0, wv1, wv2, wv3, agg_sh,
    ibs0, ibs1, gs0, gs1, gs2, gs3, ws0, ws1, ws2, ws3,
    ss0, ss1, ss2, ss3,
):
    c = lax.axis_index("c")
    s = lax.axis_index("s")
    wid = s * NC + c
    g0 = wid * 31 + jnp.minimum(wid, 8)
    cid0 = g0 * CPB
    nblk = NBLK0 + jnp.where(wid < 8, 1, 0)

    ibv = [ib0, ib1]
    gxv = [gx0, gx1, gx2, gx3]
    wfv = [wv0, wv1, wv2, wv3]
    ibsem = [ibs0, ibs1]
    gsem = [gs0, gs1, gs2, gs3]
    wsem = [ws0, ws1, ws2, ws3]
    ssem = [ss0, ss1, ss2, ss3]

    def fetch_iblk(gB, par):
        pltpu.async_copy(ei_hbm.at[gB], ibv[par], ibsem[par])

    def wait_iblk(par):
        pltpu.make_async_copy(ei_hbm.at[0], ibv[par], ibsem[par]).wait()

    def issue_gw(t, par, row, b):
        base = (cid0 + t) * CH
        pltpu.async_copy(xs_hbm.at[ibv[par].at[row]], gxv[b], gsem[b])
        pltpu.async_copy(wf_hbm.at[pl.ds(base, CH)], wfv[b], wsem[b])

    def wait_gw(b):
        pltpu.make_async_copy(xs_hbm.at[pl.ds(0, CH)], gxv[b], gsem[b]).wait()
        pltpu.make_async_copy(wf_hbm.at[pl.ds(0, CH)], wfv[b], wsem[b]).wait()

    def issue_scatter(par, row, b):
        pltpu.async_copy(
            gxv[b], agg_sh.at[ibv[par].at[CPB + row]], ssem[b], add=True
        )

    def wait_scatter(b):
        pltpu.make_async_copy(gxv[b], agg_sh.at[pl.ds(0, CH)], ssem[b]).wait()

    def multiply(b):
        def row_(r, _):
            for j in range(F // L):
                sl = (r, pl.ds(j * L, L))
                gxv[b][sl] = gxv[b][sl] * wfv[b][sl]
            return 0

        lax.fori_loop(0, CH, row_, 0)

    def zrow(r, _):
        for j in range(F // L):
            gx0[r, pl.ds(j * L, L)] = jnp.zeros((L,), jnp.float32)
        return 0

    lax.fori_loop(0, CH, zrow, 0)
    row0 = s * RPT
    for t in range(15):
        pltpu.sync_copy(gx0, agg_sh.at[pl.ds(row0 + t * CH, CH)])
    pltpu.sync_copy(
        gx0.at[pl.ds(0, 24)], agg_sh.at[pl.ds(row0 + 15 * CH, 24)]
    )

    @pl.when(s == 0)
    def _zero_rem():
        pltpu.sync_copy(
            gx0.at[pl.ds(0, NREM)], agg_sh.at[pl.ds(NS * RPT, NREM)]
        )

    plsc.subcore_barrier()

    def substep(B, Bpar, p8, first_block, last_possible):
        t = B * CPB + p8
        b = p8 % 4
        wait_gw(b)
        if (not first_block) or p8 >= 2:
            wait_scatter((p8 + 2) % 4)
        if p8 == 2 and not last_possible:
            @pl.when(B + 1 < nblk)
            def _():
                fetch_iblk(g0 + B + 1, 1 - Bpar)
        if p8 == 6 and not last_possible:
            @pl.when(B + 1 < nblk)
            def _():
                wait_iblk(1 - Bpar)
        if p8 < 6:
            if last_possible:
                issue_gw(t + 2, Bpar, p8 + 2, (p8 + 2) % 4)
            else:
                issue_gw(t + 2, Bpar, p8 + 2, (p8 + 2) % 4)
        else:
            if not last_possible:
                @pl.when(B + 1 < nblk)
                def _():
                    issue_gw(t + 2, 1 - Bpar, p8 + 2 - CPB, (p8 + 2) % 4)
        multiply(b)
        issue_scatter(Bpar, p8, b)

    fetch_iblk(g0, 0)
    wait_iblk(0)
    issue_gw(0, 0, 0, 0)
    issue_gw(1, 0, 1, 1)

    for p8 in range(CPB):
        substep(jnp.int32(0), 0, p8, True, False)

    def pair(bb, _):
        B = 1 + 2 * bb
        for p8 in range(CPB):
            substep(B, 1, p8, False, False)
        for p8 in range(CPB):
            substep(B + 1, 0, p8, False, False)
        return 0

    lax.fori_loop(0, (NBLK0 - 1) // 2, pair, 0)

    @pl.when(wid < 8)
    def _extra():
        for p8 in range(CPB):
            substep(jnp.int32(NBLK0), 1, p8, False, True)

    wait_scatter(2)
    wait_scatter(3)

    plsc.subcore_barrier()

    for t in range(15):
        r0 = row0 + t * CH
        pltpu.sync_copy(agg_sh.at[pl.ds(r0, CH)], gx0)
        pltpu.sync_copy(gx0, out_hbm.at[c, pl.ds(r0, CH)])
    pltpu.sync_copy(
        agg_sh.at[pl.ds(row0 + 15 * CH, 24)], gx0.at[pl.ds(0, 24)]
    )
    pltpu.sync_copy(
        gx0.at[pl.ds(0, 24)], out_hbm.at[c, pl.ds(row0 + 15 * CH, 24)]
    )

    @pl.when(s == 0)
    def _dump_rem():
        pltpu.sync_copy(
            agg_sh.at[pl.ds(NS * RPT, NREM)], gx1.at[pl.ds(0, NREM)]
        )
        pltpu.sync_copy(
            gx1.at[pl.ds(0, NREM)], out_hbm.at[c, pl.ds(NS * RPT, NREM)]
        )


BN = 2000


def _tail_body(p_ref, w2_ref, b2_ref, w_ref, b_ref, o_ref):
    agg = p_ref[0] + p_ref[1]
    t = _ssp(
        jnp.dot(agg, w2_ref[...], preferred_element_type=jnp.float32)
        + b2_ref[...]
    )
    o_ref[...] = (
        jnp.dot(t, w_ref[...], preferred_element_type=jnp.float32)
        + b_ref[...]
    )


def _tail_call(partials, lin2_W, lin2_br, lin_W, lin_br):
    return pl.pallas_call(
        _tail_body,
        grid=(N // BN,),
        in_specs=[
            pl.BlockSpec((NC, BN, F), lambda i: (0, i, 0)),
            pl.BlockSpec((F, H), lambda i: (0, 0)),
            pl.BlockSpec((1, H), lambda i: (0, 0)),
            pl.BlockSpec((H, H), lambda i: (0, 0)),
            pl.BlockSpec((1, H), lambda i: (0, 0)),
        ],
        out_specs=pl.BlockSpec((BN, H), lambda i: (i, 0)),
        out_shape=jax.ShapeDtypeStruct((N, H), jnp.float32),
    )(partials, lin2_W, lin2_br, lin_W, lin_br)


@jax.jit
def kernel(
    x_atom, edge_index, edge_weight, edge_attr,
    mlp_W1, mlp_b1, mlp_W2, mlp_b2,
    lin1_W, lin2_W, lin2_b, lin_W, lin_b,
):
    wf = _wf_call(
        edge_attr.T, edge_weight.reshape(E // BE, NBC, 128),
        mlp_W1, mlp_b1.reshape(1, F), mlp_W2, mlp_b2.reshape(1, F),
    )
    xs = _xs_call(x_atom, lin1_W)
    ei_r = jnp.concatenate(
        [
            edge_index[0].reshape(NEI, CPB, CH),
            edge_index[1].reshape(NEI, CPB, CH),
        ],
        axis=1,
    )
    partials = _get_sc_scatter()(xs, ei_r, wf)
    return _tail_call(
        partials, lin2_W, lin2_b.reshape(1, H), lin_W, lin_b.reshape(1, H)
    )


# speedup vs baseline: 4.9746x; 1.0012x over previous
import functools

import jax
import jax.numpy as jnp
from jax import lax
from jax.experimental import pallas as pl
from jax.experimental.pallas import tpu as pltpu
from jax.experimental.pallas import tpu_sc as plsc

N = 10000
E = 320000
H = 128
G = 50
F = 128
CUTOFF = 10.0
LOG2 = 0.6931471805599453

NC = 2
NS = 16
L = 16
NW = NC * NS
CH = 40
CPB = 8
NBLK0 = 31
NEI = E // (CH * CPB)
RPT = 624
NREM = N - NS * RPT

def _ssp(x):
    return jnp.maximum(x, 0.0) + jnp.log1p(jnp.exp(-jnp.abs(x))) - LOG2


BE = 2560
NBC = BE // 128


def _wf_body(ea_ref, ew_ref, w1_ref, b1_ref, w2_ref, b2_ref, out_ref):
    cpk = 0.5 * (jnp.cos(ew_ref[0] * (jnp.pi / CUTOFF)) + 1.0)
    ct = jnp.swapaxes(cpk, 0, 1)
    h = _ssp(
        lax.dot_general(
            ea_ref[...], w1_ref[...],
            (((0,), (0,)), ((), ())),
            preferred_element_type=jnp.float32,
        )
        + b1_ref[...]
    )
    wf = (
        jnp.dot(h, w2_ref[...], preferred_element_type=jnp.float32)
        + b2_ref[...]
    )
    for a in range(NBC):
        out_ref[a * 128:(a + 1) * 128, :] = (
            wf[a * 128:(a + 1) * 128, :] * ct[:, a:a + 1]
        )


def _wf_call(ea_t, ewp, w1, b1r, w2, b2r):
    return pl.pallas_call(
        _wf_body,
        grid=(E // BE,),
        in_specs=[
            pl.BlockSpec((G, BE), lambda i: (0, i)),
            pl.BlockSpec((1, NBC, 128), lambda i: (i, 0, 0)),
            pl.BlockSpec((G, F), lambda i: (0, 0)),
            pl.BlockSpec((1, F), lambda i: (0, 0)),
            pl.BlockSpec((F, F), lambda i: (0, 0)),
            pl.BlockSpec((1, F), lambda i: (0, 0)),
        ],
        out_specs=pl.BlockSpec((BE, F), lambda i: (i, 0)),
        out_shape=jax.ShapeDtypeStruct((E, F), jnp.float32),
    )(ea_t, ewp, w1, b1r, w2, b2r)


def _xs_body(x_ref, w_ref, o_ref):
    o_ref[...] = jnp.dot(
        x_ref[...], w_ref[...], preferred_element_type=jnp.float32
    )


def _xs_call(x_atom, lin1_W):
    return pl.pallas_call(
        _xs_body,
        out_shape=jax.ShapeDtypeStruct((N, F), jnp.float32),
    )(x_atom, lin1_W)


@functools.cache
def _get_sc_scatter():
    mesh = plsc.VectorSubcoreMesh(
        core_axis_name="c", subcore_axis_name="s",
        num_cores=NC, num_subcores=NS,
    )
    return functools.partial(
        pl.kernel,
        mesh=mesh,
        out_type=jax.ShapeDtypeStruct((NC, N, F), jnp.float32),
        scratch_types=[
            pltpu.VMEM((2 * CPB, CH), jnp.int32),
            pltpu.VMEM((2 * CPB, CH), jnp.int32),
            pltpu.VMEM((CH, F), jnp.float32),
            pltpu.VMEM((CH, F), jnp.float32),
            pltpu.VMEM((CH, F), jnp.float32),
            pltpu.VMEM((CH, F), jnp.float32),
            pltpu.VMEM((CH, F), jnp.float32),
            pltpu.VMEM((CH, F), jnp.float32),
            pltpu.VMEM((CH, F), jnp.float32),
            pltpu.VMEM((CH, F), jnp.float32),
            pltpu.VMEM_SHARED((N, F), jnp.float32),
            pltpu.SemaphoreType.DMA,
            pltpu.SemaphoreType.DMA,
            pltpu.SemaphoreType.DMA,
            pltpu.SemaphoreType.DMA,
            pltpu.SemaphoreType.DMA,
            pltpu.SemaphoreType.DMA,
            pltpu.SemaphoreType.DMA,
            pltpu.SemaphoreType.DMA,
            pltpu.SemaphoreType.DMA,
            pltpu.SemaphoreType.DMA,
            pltpu.SemaphoreType.DMA,
            pltpu.SemaphoreType.DMA,
            pltpu.SemaphoreType.DMA,
            pltpu.SemaphoreType.DMA,
        ],
    )(_sc_scatter_body)


def _sc_scatter_body(
    xs_hbm, ei_hbm, wf_hbm, out_hbm,
    ib0, ib1, gx0, gx1, gx2, gx3, wv0, wv1, wv2, wv3, agg_sh,
    ibs0, ibs1, gs0, gs1, gs2, gs3, ws0, ws1, ws2, ws3,
    ss0, ss1, ss2, ss3,
):
    c = lax.axis_index("c")
    s = lax.axis_index("s")
    wid = s * NC + c
    g0 = wid * 31 + jnp.minimum(wid, 8)
    cid0 = g0 * CPB
    nblk = NBLK0 + jnp.where(wid < 8, 1, 0)

    ibv = [ib0, ib1]
    gxv = [gx0, gx1, gx2, gx3]
    wfv = [wv0, wv1, wv2, wv3]
    ibsem = [ibs0, ibs1]
    gsem = [gs0, gs1, gs2, gs3]
    wsem = [ws0, ws1, ws2, ws3]
    ssem = [ss0, ss1, ss2, ss3]

    def fetch_iblk(gB, par):
        pltpu.async_copy(ei_hbm.at[gB], ibv[par], ibsem[par])

    def wait_iblk(par):
        pltpu.make_async_copy(ei_hbm.at[0], ibv[par], ibsem[par]).wait()

    def issue_gw(t, par, row, b):
        base = (cid0 + t) * CH
        pltpu.async_copy(xs_hbm.at[ibv[par].at[row]], gxv[b], gsem[b])
        pltpu.async_copy(wf_hbm.at[pl.ds(base, CH)], wfv[b], wsem[b])

    def wait_gw(b):
        pltpu.make_async_copy(xs_hbm.at[pl.ds(0, CH)], gxv[b], gsem[b]).wait()
        pltpu.make_async_copy(wf_hbm.at[pl.ds(0, CH)], wfv[b], wsem[b]).wait()

    def issue_scatter(par, row, b):
        pltpu.async_copy(
            gxv[b], agg_sh.at[ibv[par].at[CPB + row]], ssem[b], add=True
        )

    def wait_scatter(b):
        pltpu.make_async_copy(gxv[b], agg_sh.at[pl.ds(0, CH)], ssem[b]).wait()

    def multiply(b):
        def row_(r, _):
            for j in range(F // L):
                sl = (r, pl.ds(j * L, L))
                gxv[b][sl] = gxv[b][sl] * wfv[b][sl]
            return 0

        lax.fori_loop(0, CH, row_, 0)

    def zrow(r, _):
        for j in range(F // L):
            gx0[r, pl.ds(j * L, L)] = jnp.zeros((L,), jnp.float32)
        return 0

    lax.fori_loop(0, CH, zrow, 0)
    row0 = s * RPT
    for t in range(15):
        pltpu.sync_copy(gx0, agg_sh.at[pl.ds(row0 + t * CH, CH)])
    pltpu.sync_copy(
        gx0.at[pl.ds(0, 24)], agg_sh.at[pl.ds(row0 + 15 * CH, 24)]
    )

    @pl.when(s == 0)
    def _zero_rem():
        pltpu.sync_copy(
            gx0.at[pl.ds(0, NREM)], agg_sh.at[pl.ds(NS * RPT, NREM)]
        )

    plsc.subcore_barrier()

    def substep(B, Bpar, p8, first_block, last_possible):
        t = B * CPB + p8
        b = p8 % 4
        wait_gw(b)
        if (not first_block) or p8 >= 2:
            wait_scatter((p8 + 2) % 4)
        if p8 == 2 and not last_possible:
            @pl.when(B + 1 < nblk)
            def _():
                fetch_iblk(g0 + B + 1, 1 - Bpar)
        if p8 == 6 and not last_possible:
            @pl.when(B + 1 < nblk)
            def _():
                wait_iblk(1 - Bpar)
        if p8 < 6:
            issue_gw(t + 2, Bpar, p8 + 2, (p8 + 2) % 4)
        else:
            if not last_possible:
                @pl.when(B + 1 < nblk)
                def _():
                    issue_gw(t + 2, 1 - Bpar, p8 + 2 - CPB, (p8 + 2) % 4)
        multiply(b)
        issue_scatter(Bpar, p8, b)

    fetch_iblk(g0, 0)
    wait_iblk(0)
    issue_gw(0, 0, 0, 0)
    issue_gw(1, 0, 1, 1)

    for p8 in range(CPB):
        substep(jnp.int32(0), 0, p8, True, False)

    def pair(bb, _):
        B = 1 + 2 * bb
        for p8 in range(CPB):
            substep(B, 1, p8, False, False)
        for p8 in range(CPB):
            substep(B + 1, 0, p8, False, False)
        return 0

    lax.fori_loop(0, (NBLK0 - 1) // 2, pair, 0)

    @pl.when(wid < 8)
    def _extra():
        for p8 in range(CPB):
            substep(jnp.int32(NBLK0), 1, p8, False, True)

    wait_scatter(2)
    wait_scatter(3)

    plsc.subcore_barrier()

    for t in range(15):
        r0 = row0 + t * CH
        pltpu.sync_copy(agg_sh.at[pl.ds(r0, CH)], gx0)
        pltpu.sync_copy(gx0, out_hbm.at[c, pl.ds(r0, CH)])
    pltpu.sync_copy(
        agg_sh.at[pl.ds(row0 + 15 * CH, 24)], gx0.at[pl.ds(0, 24)]
    )
    pltpu.sync_copy(
        gx0.at[pl.ds(0, 24)], out_hbm.at[c, pl.ds(row0 + 15 * CH, 24)]
    )

    @pl.when(s == 0)
    def _dump_rem():
        pltpu.sync_copy(
            agg_sh.at[pl.ds(NS * RPT, NREM)], gx1.at[pl.ds(0, NREM)]
        )
        pltpu.sync_copy(
            gx1.at[pl.ds(0, NREM)], out_hbm.at[c, pl.ds(NS * RPT, NREM)]
        )


BN = 2000


def _tail_body(p_ref, w2_ref, b2_ref, w_ref, b_ref, o_ref):
    agg = p_ref[0] + p_ref[1]
    t = _ssp(
        jnp.dot(agg, w2_ref[...], preferred_element_type=jnp.float32)
        + b2_ref[...]
    )
    o_ref[...] = (
        jnp.dot(t, w_ref[...], preferred_element_type=jnp.float32)
        + b_ref[...]
    )


def _tail_call(partials, lin2_W, lin2_br, lin_W, lin_br):
    return pl.pallas_call(
        _tail_body,
        grid=(N // BN,),
        in_specs=[
            pl.BlockSpec((NC, BN, F), lambda i: (0, i, 0)),
            pl.BlockSpec((F, H), lambda i: (0, 0)),
            pl.BlockSpec((1, H), lambda i: (0, 0)),
            pl.BlockSpec((H, H), lambda i: (0, 0)),
            pl.BlockSpec((1, H), lambda i: (0, 0)),
        ],
        out_specs=pl.BlockSpec((BN, H), lambda i: (i, 0)),
        out_shape=jax.ShapeDtypeStruct((N, H), jnp.float32),
    )(partials, lin2_W, lin2_br, lin_W, lin_br)


@jax.jit
def kernel(
    x_atom, edge_index, edge_weight, edge_attr,
    mlp_W1, mlp_b1, mlp_W2, mlp_b2,
    lin1_W, lin2_W, lin2_b, lin_W, lin_b,
):
    wf = _wf_call(
        edge_attr.T, edge_weight.reshape(E // BE, NBC, 128),
        mlp_W1, mlp_b1.reshape(1, F), mlp_W2, mlp_b2.reshape(1, F),
    )
    xs = _xs_call(x_atom, lin1_W)
    ei_r = jnp.concatenate(
        [
            edge_index[0].reshape(NEI, CPB, CH),
            edge_index[1].reshape(NEI, CPB, CH),
        ],
        axis=1,
    )
    partials = _get_sc_scatter()(xs, ei_r, wf)
    return _tail_call(
        partials, lin2_W, lin2_b.reshape(1, H), lin_W, lin_b.reshape(1, H)
    )
